# Initial kernel scaffold; baseline (speedup 1.0000x reference)
import jax
import jax.numpy as jnp
from jax.experimental import pallas as pl


def kernel(img_feats, verts_padded, edges_packed, vert_feats_prev, sem_2d, P, bneck_w, bneck_b, vsem_w, vsem_b, g0_w0, g0_b0, g0_w1, g0_b1, g1_w0, g1_b0, g1_w1, g1_b1, g2_w0, g2_b0, g2_w1, g2_b1):
    raise NotImplementedError("write your pallas kernel here")


# speedup vs baseline: 1.1623x; 1.1623x over previous
import jax
import jax.numpy as jnp
from jax import lax
from jax.experimental import pallas as pl
from jax.experimental.pallas import tpu as pltpu
from jax.experimental.pallas import tpu_sc as plsc

N, V, IMG_DIM, HID, VFD, NC, H, W = 4, 10000, 512, 512, 128, 32, 64, 64
NV = N * V
PIX = H * W
TD = HID + NC
FC = HID // NC
NVP = 40960
VPW = NVP // 32
SB = 32
SBT = VPW // SB
E = 120000
NE2 = 2 * E
EB = 128
EBT = 118
NE2P = 16 * EB * EBT
DUMMY = NVP - 8
CW = 16
NCH = HID // CW
ZR = NVP // 16
EPB = 128

def _mesh():
    return plsc.VectorSubcoreMesh(core_axis_name="c", subcore_axis_name="s",
                                  num_cores=2, num_subcores=16)


def _prep_body(vt_ref, idx_ref, w_ref):
    n = pl.program_id(0)
    gx = (vt_ref[0, 0] + 1.0) * 0.5 * (W - 1)
    gy = (vt_ref[0, 1] + 1.0) * 0.5 * (H - 1)
    x0 = jnp.floor(gx)
    x1 = x0 + 1.0
    y0 = jnp.floor(gy)
    y1 = y0 + 1.0
    wx1 = gx - x0
    wx0 = 1.0 - wx1
    wy1 = gy - y0
    wy0 = 1.0 - wy1
    base = n * PIX
    taps = [(x0, y0, wx0 * wy0), (x1, y0, wx1 * wy0),
            (x0, y1, wx0 * wy1), (x1, y1, wx1 * wy1)]
    for t, (xi, yi, wt) in enumerate(taps):
        valid = ((xi >= 0) & (xi <= W - 1) & (yi >= 0) & (yi <= H - 1))
        xci = jnp.clip(xi, 0, W - 1).astype(jnp.int32)
        yci = jnp.clip(yi, 0, H - 1).astype(jnp.int32)
        idx_ref[0, t] = base + yci * W + xci
        w_ref[0, t] = wt * valid.astype(jnp.float32)


def _prep_call(verts_t):
    return pl.pallas_call(
        _prep_body,
        grid=(N,),
        in_specs=[
            pl.BlockSpec((1, 3, V), lambda n: (n, 0, 0)),
        ],
        out_specs=[
            pl.BlockSpec((1, 4, V), lambda n: (n, 0, 0)),
            pl.BlockSpec((1, 4, V), lambda n: (n, 0, 0)),
        ],
        out_shape=[
            jax.ShapeDtypeStruct((N, 4, V), jnp.int32),
            jax.ShapeDtypeStruct((N, 4, V), jnp.float32),
        ],
    )(verts_t)


def _pix_body(a_ref, w_ref, o_ref):
    o_ref[...] = jnp.dot(a_ref[...], w_ref[...],
                         preferred_element_type=jnp.float32)


def _pix_call(img_t, wt):
    m = N * PIX
    return pl.pallas_call(
        _pix_body,
        grid=(m // 512,),
        in_specs=[
            pl.BlockSpec((512, IMG_DIM), lambda i: (i, 0)),
            pl.BlockSpec((IMG_DIM, HID), lambda i: (0, 0)),
        ],
        out_specs=pl.BlockSpec((512, HID), lambda i: (i, 0)),
        out_shape=jax.ShapeDtypeStruct((m, HID), jnp.float32),
    )(img_t, wt)


def _sample_body(tap_idx, tap_w, table, bias, out, idxb, wb, rows, outb,
                 biasb, sem):
    cid = lax.axis_index("c")
    sid = lax.axis_index("s")
    wid = sid * 2 + cid
    bbase = wid * SBT
    pltpu.sync_copy(bias, biasb)

    @pl.loop(0, SBT)
    def _batch(b):
        bg = bbase + b
        pltpu.sync_copy(tap_idx.at[bg], idxb)
        pltpu.sync_copy(tap_w.at[bg], wb)
        for t in range(4):
            pltpu.async_copy(table.at[idxb.at[pl.ds(t * SB, SB)]],
                             rows.at[t], sem).wait()

        @pl.loop(0, SB)
        def _v(v):
            w0 = wb[pl.ds(v, 16)][0]
            w1 = wb[pl.ds(v + SB, 16)][0]
            w2 = wb[pl.ds(v + 2 * SB, 16)][0]
            w3 = wb[pl.ds(v + 3 * SB, 16)][0]
            for c in range(TD // 16):
                sl = pl.ds(c * 16, 16)
                acc = (rows[0, v, sl] * w0 + rows[1, v, sl] * w1
                       + rows[2, v, sl] * w2 + rows[3, v, sl] * w3)
                if c < HID // 16:
                    acc = jnp.maximum(acc + biasb[sl], 0.0)
                outb[v, sl] = acc

        pltpu.sync_copy(outb, out.at[pl.ds(bg * SB, SB)])


def _sample_call(tap_idx, tap_w, table, bias):
    return pl.kernel(
        _sample_body,
        out_type=jax.ShapeDtypeStruct((NVP, TD), jnp.float32),
        mesh=_mesh(),
        compiler_params=pltpu.CompilerParams(use_tc_tiling_on_sc=False),
        scratch_types=[
            pltpu.VMEM((4 * SB,), jnp.int32),
            pltpu.VMEM((4 * SB + SB,), jnp.float32),
            pltpu.VMEM((4, SB, TD), jnp.float32),
            pltpu.VMEM((SB, TD), jnp.float32),
            pltpu.VMEM((HID,), jnp.float32),
            pltpu.SemaphoreType.DMA,
        ],
    )(tap_idx, tap_w, table, bias)


def _agg_body(srcs2, dsts2, vw1r, vw0r, h_out, srcb, dstb, srcadj,
              rows, abuf, bbuf, idxe, zbuf, acc, sem):
    cid = lax.axis_index("c")
    sid = lax.axis_index("s")
    pltpu.sync_copy(srcs2.at[pl.ds(sid * EBT, EBT)], srcb)
    pltpu.sync_copy(dsts2.at[pl.ds(sid * EBT, EBT)], dstb)
    zv = jnp.zeros((16,), jnp.float32)

    @pl.loop(0, EPB)
    def _z(i):
        zbuf[i] = zv

    @pl.loop(0, NCH // 2)
    def _chunk(cc):
        chunk = cid * (NCH // 2) + cc
        chunkv = jnp.zeros((16,), jnp.int32) + chunk

        @pl.loop(0, ZR // EPB)
        def _zero(p):
            pltpu.sync_copy(zbuf, acc.at[pl.ds(sid * ZR + p * EPB, EPB)])

        @pl.loop(0, EBT)
        def _adj(j):
            for k in range(EB // 16):
                sl = pl.ds(k * 16, 16)
                srcadj[j, sl] = srcb[j, sl] * NCH + chunkv

        plsc.subcore_barrier()

        @pl.loop(0, EBT)
        def _edge(j):
            pltpu.async_copy(vw1r.at[srcadj.at[j]], rows, sem).wait()
            pltpu.sync_copy(rows, acc.at[dstb.at[j]], add=True)

        plsc.subcore_barrier()

        @pl.loop(0, ZR // EPB)
        def _ep(p):
            r0 = sid * ZR + p * EPB
            for k in range(EPB // 16):
                idxe[pl.ds(k * 16, 16)] = (
                    (jnp.zeros((16,), jnp.int32) + (r0 + k * 16)
                     + lax.iota(jnp.int32, 16)) * NCH + chunkv)
            pltpu.sync_copy(acc.at[pl.ds(r0, EPB)], abuf)
            pltpu.async_copy(vw0r.at[idxe], bbuf, sem).wait()

            @pl.loop(0, EPB)
            def _h(i):
                abuf[i] = jnp.maximum(abuf[i] + bbuf[i], 0.0)

            pltpu.sync_copy(abuf, h_out.at[idxe])

        plsc.subcore_barrier()


def _agg_call(srcs2, dsts2, vw1r, vw0r):
    return pl.kernel(
        _agg_body,
        out_type=jax.ShapeDtypeStruct((NVP * NCH, CW), jnp.float32),
        mesh=_mesh(),
        compiler_params=pltpu.CompilerParams(use_tc_tiling_on_sc=False),
        scratch_types=[
            pltpu.VMEM((EBT, EB), jnp.int32),
            pltpu.VMEM((EBT, EB), jnp.int32),
            pltpu.VMEM((EBT, EB), jnp.int32),
            pltpu.VMEM((EB, CW), jnp.float32),
            pltpu.VMEM((EPB, CW), jnp.float32),
            pltpu.VMEM((EPB, CW), jnp.float32),
            pltpu.VMEM((EPB,), jnp.int32),
            pltpu.VMEM((EPB, CW), jnp.float32),
            pltpu.VMEM_SHARED((NVP, CW), jnp.float32),
            pltpu.SemaphoreType.DMA,
        ],
    )(srcs2, dsts2, vw1r, vw0r)


def _mm2_body(a_ref, w0_ref, w1_ref, b0_ref, b1_ref, o0_ref, o1_ref):
    a = a_ref[...]
    o0_ref[...] = (jnp.dot(a, w0_ref[...], preferred_element_type=jnp.float32)
                   + b0_ref[...])
    o1_ref[...] = (jnp.dot(a, w1_ref[...], preferred_element_type=jnp.float32)
                   + b1_ref[...])


def _mm2_call(x, w0t, w1t, b0, b1):
    k = x.shape[1]
    return pl.pallas_call(
        _mm2_body,
        grid=(NVP // 512,),
        in_specs=[
            pl.BlockSpec((512, k), lambda i: (i, 0)),
            pl.BlockSpec((k, HID), lambda i: (0, 0)),
            pl.BlockSpec((k, HID), lambda i: (0, 0)),
            pl.BlockSpec((1, HID), lambda i: (0, 0)),
            pl.BlockSpec((1, HID), lambda i: (0, 0)),
        ],
        out_specs=[
            pl.BlockSpec((512, HID), lambda i: (i, 0)),
            pl.BlockSpec((512, HID), lambda i: (i, 0)),
        ],
        out_shape=[
            jax.ShapeDtypeStruct((NVP, HID), jnp.float32),
            jax.ShapeDtypeStruct((NVP, HID), jnp.float32),
        ],
    )(x, w0t, w1t, b0, b1)


def _fin_body(a_ref, w_ref, b_ref, t_ref, o_ref):
    o_ref[...] = (jnp.dot(a_ref[...], w_ref[...],
                          preferred_element_type=jnp.float32)
                  + b_ref[...] + t_ref[...])


def _fin_call(x, wt, b, tex):
    k = x.shape[1]
    return pl.pallas_call(
        _fin_body,
        grid=(NVP // 512,),
        in_specs=[
            pl.BlockSpec((512, k), lambda i: (i, 0)),
            pl.BlockSpec((k, NC), lambda i: (0, 0)),
            pl.BlockSpec((1, NC), lambda i: (0, 0)),
            pl.BlockSpec((512, NC), lambda i: (i, 0)),
        ],
        out_specs=pl.BlockSpec((512, NC), lambda i: (i, 0)),
        out_shape=jax.ShapeDtypeStruct((NVP, NC), jnp.float32),
    )(x, wt, b, tex)


def kernel(img_feats, verts_padded, edges_packed, vert_feats_prev, sem_2d, P,
           bneck_w, bneck_b, vsem_w, vsem_b,
           g0_w0, g0_b0, g0_w1, g0_b1,
           g1_w0, g1_b0, g1_w1, g1_b1,
           g2_w0, g2_b0, g2_w1, g2_b1):
    ones = jnp.ones((N, V, 1), verts_padded.dtype)
    verts_hom = jnp.concatenate([verts_padded, ones], axis=2)
    verts_cam = jnp.einsum('nvi,nji->nvj', verts_hom, P)
    wcl = verts_cam[:, :, 3:]
    eps = 1e-1
    wcl = jnp.where(jnp.abs(wcl) < eps, jnp.where(wcl >= 0, eps, -eps), wcl)
    verts_t = (verts_cam[:, :, :3] / wcl).transpose(0, 2, 1)
    img_t = img_feats.transpose(0, 2, 3, 1).reshape(N * PIX, IMG_DIM)
    sem_t = sem_2d.transpose(0, 2, 3, 1).reshape(N * PIX, NC)
    pos = verts_padded.reshape(NV, 3)
    pos_pad = jnp.pad(pos, ((0, NVP - NV), (0, 0)))
    prev_pad = jnp.pad(vert_feats_prev, ((0, NVP - NV), (0, 0)))

    srcs = jnp.concatenate([edges_packed[:, 1], edges_packed[:, 0]])
    dsts = jnp.concatenate([edges_packed[:, 0], edges_packed[:, 1]])
    srcs2 = jnp.pad(srcs, (0, NE2P - NE2)).reshape(16 * EBT, EB)
    dsts2 = jnp.concatenate(
        [dsts, jnp.full((NE2P - NE2,), DUMMY, jnp.int32)]).reshape(
            16 * EBT, EB)

    idx4, w4 = _prep_call(verts_t)
    tap_idx = jnp.pad(idx4.transpose(1, 0, 2).reshape(4, NV),
                      ((0, 0), (0, NVP - NV)))
    tap_w = jnp.pad(w4.transpose(1, 0, 2).reshape(4, NV),
                    ((0, 0), (0, NVP - NV)))
    tap_idx = tap_idx.reshape(4, NVP // SB, SB).transpose(1, 0, 2).reshape(
        NVP // SB, 4 * SB)
    tap_w = tap_w.reshape(4, NVP // SB, SB).transpose(1, 0, 2).reshape(
        NVP // SB, 4 * SB)
    tap_w = jnp.pad(tap_w, ((0, 0), (0, SB)))

    pp = _pix_call(img_t, bneck_w.T)
    table = jnp.concatenate([pp, sem_t], axis=1)

    st = _sample_call(tap_idx, tap_w, table, bneck_b)
    va_sem = st[:, :HID]
    tex = st[:, HID:]

    x = jnp.concatenate([va_sem, pos_pad, tex, prev_pad], axis=1)
    h = None
    for (w0, b0, w1, b1) in ((g0_w0, g0_b0, g0_w1, g0_b1),
                             (g1_w0, g1_b0, g1_w1, g1_b1),
                             (g2_w0, g2_b0, g2_w1, g2_b1)):
        vw0, vw1 = _mm2_call(x, w0.T, w1.T, b0[None], b1[None])
        h = _agg_call(srcs2, dsts2, vw1.reshape(NVP * NCH, CW),
                      vw0.reshape(NVP * NCH, CW)).reshape(NVP, HID)
        x = jnp.concatenate([h, pos_pad, tex], axis=1)

    finp = _fin_call(x, vsem_w.T, vsem_b[None], tex)
    final_textures = finp[:NV].reshape(-1, V, NC)
    return final_textures, h[:NV]


# speedup vs baseline: 1.6564x; 1.4251x over previous
import jax
import jax.numpy as jnp
from jax import lax
from jax.experimental import pallas as pl
from jax.experimental.pallas import tpu as pltpu
from jax.experimental.pallas import tpu_sc as plsc

N, V, IMG_DIM, HID, VFD, NC, H, W = 4, 10000, 512, 512, 128, 32, 64, 64
NV = N * V
PIX = H * W
TD = HID + NC
FC = HID // NC
NVP = 40960
VPW = NVP // 32
SB = 32
SBT = VPW // SB
E = 120000
NE2 = 2 * E
EB = 128
EBT = 120
NB = 8
NE2P = 16 * EB * EBT
DUMMY = NVP - 8
CW = 16
NCH = HID // CW
ZR = NVP // 16
EPB = 128

def _mesh():
    return plsc.VectorSubcoreMesh(core_axis_name="c", subcore_axis_name="s",
                                  num_cores=2, num_subcores=16)


def _prep_body(vt_ref, idx_ref, w_ref):
    n = pl.program_id(0)
    gx = (vt_ref[0, 0] + 1.0) * 0.5 * (W - 1)
    gy = (vt_ref[0, 1] + 1.0) * 0.5 * (H - 1)
    x0 = jnp.floor(gx)
    x1 = x0 + 1.0
    y0 = jnp.floor(gy)
    y1 = y0 + 1.0
    wx1 = gx - x0
    wx0 = 1.0 - wx1
    wy1 = gy - y0
    wy0 = 1.0 - wy1
    base = n * PIX
    taps = [(x0, y0, wx0 * wy0), (x1, y0, wx1 * wy0),
            (x0, y1, wx0 * wy1), (x1, y1, wx1 * wy1)]
    for t, (xi, yi, wt) in enumerate(taps):
        valid = ((xi >= 0) & (xi <= W - 1) & (yi >= 0) & (yi <= H - 1))
        xci = jnp.clip(xi, 0, W - 1).astype(jnp.int32)
        yci = jnp.clip(yi, 0, H - 1).astype(jnp.int32)
        idx_ref[0, t] = base + yci * W + xci
        w_ref[0, t] = wt * valid.astype(jnp.float32)


def _prep_call(verts_t):
    return pl.pallas_call(
        _prep_body,
        grid=(N,),
        in_specs=[
            pl.BlockSpec((1, 3, V), lambda n: (n, 0, 0)),
        ],
        out_specs=[
            pl.BlockSpec((1, 4, V), lambda n: (n, 0, 0)),
            pl.BlockSpec((1, 4, V), lambda n: (n, 0, 0)),
        ],
        out_shape=[
            jax.ShapeDtypeStruct((N, 4, V), jnp.int32),
            jax.ShapeDtypeStruct((N, 4, V), jnp.float32),
        ],
    )(verts_t)


def _pix_body(a_ref, w_ref, o_ref):
    o_ref[...] = jnp.dot(a_ref[...], w_ref[...],
                         preferred_element_type=jnp.float32)


def _pix_call(img_t, wt):
    m = N * PIX
    return pl.pallas_call(
        _pix_body,
        grid=(m // 512,),
        in_specs=[
            pl.BlockSpec((512, IMG_DIM), lambda i: (i, 0)),
            pl.BlockSpec((IMG_DIM, HID), lambda i: (0, 0)),
        ],
        out_specs=pl.BlockSpec((512, HID), lambda i: (i, 0)),
        out_shape=jax.ShapeDtypeStruct((m, HID), jnp.float32),
    )(img_t, wt)


def _sample_body(tap_idx, tap_w, table, bias, out, idxb, wb, rows, outb,
                 biasb, sem):
    cid = lax.axis_index("c")
    sid = lax.axis_index("s")
    wid = sid * 2 + cid
    bbase = wid * SBT
    pltpu.sync_copy(bias, biasb)

    @pl.loop(0, SBT)
    def _batch(b):
        bg = bbase + b
        pltpu.sync_copy(tap_idx.at[bg], idxb)
        pltpu.sync_copy(tap_w.at[bg], wb)
        for t in range(4):
            pltpu.async_copy(table.at[idxb.at[pl.ds(t * SB, SB)]],
                             rows.at[t], sem).wait()

        @pl.loop(0, SB)
        def _v(v):
            w0 = wb[pl.ds(v, 16)][0]
            w1 = wb[pl.ds(v + SB, 16)][0]
            w2 = wb[pl.ds(v + 2 * SB, 16)][0]
            w3 = wb[pl.ds(v + 3 * SB, 16)][0]
            for c in range(TD // 16):
                sl = pl.ds(c * 16, 16)
                acc = (rows[0, v, sl] * w0 + rows[1, v, sl] * w1
                       + rows[2, v, sl] * w2 + rows[3, v, sl] * w3)
                if c < HID // 16:
                    acc = jnp.maximum(acc + biasb[sl], 0.0)
                outb[v, sl] = acc

        pltpu.sync_copy(outb, out.at[pl.ds(bg * SB, SB)])


def _sample_call(tap_idx, tap_w, table, bias):
    return pl.kernel(
        _sample_body,
        out_type=jax.ShapeDtypeStruct((NVP, TD), jnp.float32),
        mesh=_mesh(),
        compiler_params=pltpu.CompilerParams(use_tc_tiling_on_sc=False),
        scratch_types=[
            pltpu.VMEM((4 * SB,), jnp.int32),
            pltpu.VMEM((4 * SB + SB,), jnp.float32),
            pltpu.VMEM((4, SB, TD), jnp.float32),
            pltpu.VMEM((SB, TD), jnp.float32),
            pltpu.VMEM((HID,), jnp.float32),
            pltpu.SemaphoreType.DMA,
        ],
    )(tap_idx, tap_w, table, bias)


def _agg_body(srcs2, dsts2, vw1r, vw0r, h_out, srcb, dstb, srcadj,
              rows, abuf, bbuf, idxe, zbuf, acc, sem):
    cid = lax.axis_index("c")
    sid = lax.axis_index("s")
    pltpu.sync_copy(srcs2.at[pl.ds(sid * EBT, EBT)], srcb)
    pltpu.sync_copy(dsts2.at[pl.ds(sid * EBT, EBT)], dstb)
    zv = jnp.zeros((16,), jnp.float32)

    @pl.loop(0, EPB)
    def _z(i):
        zbuf[i] = zv

    @pl.loop(0, NCH // 2)
    def _chunk(cc):
        chunk = cid * (NCH // 2) + cc
        chunkv = jnp.zeros((16,), jnp.int32) + chunk

        @pl.loop(0, ZR // EPB)
        def _zero(p):
            pltpu.sync_copy(zbuf, acc.at[pl.ds(sid * ZR + p * EPB, EPB)])

        @pl.loop(0, EBT)
        def _adj(j):
            for k in range(EB // 16):
                sl = pl.ds(k * 16, 16)
                srcadj[j, sl] = srcb[j, sl] * NCH + chunkv

        plsc.subcore_barrier()

        @pl.loop(0, EBT // NB)
        def _grp(g):
            j0 = g * NB
            gs = [pltpu.async_copy(vw1r.at[srcadj.at[j0 + b]], rows.at[b], sem)
                  for b in range(NB)]
            for h in gs:
                h.wait()
            ss = [pltpu.async_copy(rows.at[b], acc.at[dstb.at[j0 + b]], sem,
                                   add=True)
                  for b in range(NB)]
            for h in ss:
                h.wait()

        plsc.subcore_barrier()

        @pl.loop(0, ZR // EPB)
        def _ep(p):
            r0 = sid * ZR + p * EPB
            for k in range(EPB // 16):
                idxe[pl.ds(k * 16, 16)] = (
                    (jnp.zeros((16,), jnp.int32) + (r0 + k * 16)
                     + lax.iota(jnp.int32, 16)) * NCH + chunkv)
            pltpu.sync_copy(acc.at[pl.ds(r0, EPB)], abuf)
            pltpu.async_copy(vw0r.at[idxe], bbuf, sem).wait()

            @pl.loop(0, EPB)
            def _h(i):
                abuf[i] = jnp.maximum(abuf[i] + bbuf[i], 0.0)

            pltpu.sync_copy(abuf, h_out.at[idxe])

        plsc.subcore_barrier()


def _agg_call(srcs2, dsts2, vw1r, vw0r):
    return pl.kernel(
        _agg_body,
        out_type=jax.ShapeDtypeStruct((NVP * NCH, CW), jnp.float32),
        mesh=_mesh(),
        compiler_params=pltpu.CompilerParams(use_tc_tiling_on_sc=False),
        scratch_types=[
            pltpu.VMEM((EBT, EB), jnp.int32),
            pltpu.VMEM((EBT, EB), jnp.int32),
            pltpu.VMEM((EBT, EB), jnp.int32),
            pltpu.VMEM((NB, EB, CW), jnp.float32),
            pltpu.VMEM((EPB, CW), jnp.float32),
            pltpu.VMEM((EPB, CW), jnp.float32),
            pltpu.VMEM((EPB,), jnp.int32),
            pltpu.VMEM((EPB, CW), jnp.float32),
            pltpu.VMEM_SHARED((NVP, CW), jnp.float32),
            pltpu.SemaphoreType.DMA,
        ],
    )(srcs2, dsts2, vw1r, vw0r)


def _mm2_body(a_ref, w0_ref, w1_ref, b0_ref, b1_ref, o0_ref, o1_ref):
    a = a_ref[...]
    o0_ref[...] = (jnp.dot(a, w0_ref[...], preferred_element_type=jnp.float32)
                   + b0_ref[...])
    o1_ref[...] = (jnp.dot(a, w1_ref[...], preferred_element_type=jnp.float32)
                   + b1_ref[...])


def _mm2_call(x, w0t, w1t, b0, b1):
    k = x.shape[1]
    return pl.pallas_call(
        _mm2_body,
        grid=(NVP // 512,),
        in_specs=[
            pl.BlockSpec((512, k), lambda i: (i, 0)),
            pl.BlockSpec((k, HID), lambda i: (0, 0)),
            pl.BlockSpec((k, HID), lambda i: (0, 0)),
            pl.BlockSpec((1, HID), lambda i: (0, 0)),
            pl.BlockSpec((1, HID), lambda i: (0, 0)),
        ],
        out_specs=[
            pl.BlockSpec((512, HID), lambda i: (i, 0)),
            pl.BlockSpec((512, HID), lambda i: (i, 0)),
        ],
        out_shape=[
            jax.ShapeDtypeStruct((NVP, HID), jnp.float32),
            jax.ShapeDtypeStruct((NVP, HID), jnp.float32),
        ],
    )(x, w0t, w1t, b0, b1)


def _fin_body(a_ref, w_ref, b_ref, t_ref, o_ref):
    o_ref[...] = (jnp.dot(a_ref[...], w_ref[...],
                          preferred_element_type=jnp.float32)
                  + b_ref[...] + t_ref[...])


def _fin_call(x, wt, b, tex):
    k = x.shape[1]
    return pl.pallas_call(
        _fin_body,
        grid=(NVP // 512,),
        in_specs=[
            pl.BlockSpec((512, k), lambda i: (i, 0)),
            pl.BlockSpec((k, NC), lambda i: (0, 0)),
            pl.BlockSpec((1, NC), lambda i: (0, 0)),
            pl.BlockSpec((512, NC), lambda i: (i, 0)),
        ],
        out_specs=pl.BlockSpec((512, NC), lambda i: (i, 0)),
        out_shape=jax.ShapeDtypeStruct((NVP, NC), jnp.float32),
    )(x, wt, b, tex)


def kernel(img_feats, verts_padded, edges_packed, vert_feats_prev, sem_2d, P,
           bneck_w, bneck_b, vsem_w, vsem_b,
           g0_w0, g0_b0, g0_w1, g0_b1,
           g1_w0, g1_b0, g1_w1, g1_b1,
           g2_w0, g2_b0, g2_w1, g2_b1):
    ones = jnp.ones((N, V, 1), verts_padded.dtype)
    verts_hom = jnp.concatenate([verts_padded, ones], axis=2)
    verts_cam = jnp.einsum('nvi,nji->nvj', verts_hom, P)
    wcl = verts_cam[:, :, 3:]
    eps = 1e-1
    wcl = jnp.where(jnp.abs(wcl) < eps, jnp.where(wcl >= 0, eps, -eps), wcl)
    verts_t = (verts_cam[:, :, :3] / wcl).transpose(0, 2, 1)
    img_t = img_feats.transpose(0, 2, 3, 1).reshape(N * PIX, IMG_DIM)
    sem_t = sem_2d.transpose(0, 2, 3, 1).reshape(N * PIX, NC)
    pos = verts_padded.reshape(NV, 3)
    pos_pad = jnp.pad(pos, ((0, NVP - NV), (0, 0)))
    prev_pad = jnp.pad(vert_feats_prev, ((0, NVP - NV), (0, 0)))

    srcs = jnp.concatenate([edges_packed[:, 1], edges_packed[:, 0]])
    dsts = jnp.concatenate([edges_packed[:, 0], edges_packed[:, 1]])
    npad = NE2P - NE2
    srcs2 = jnp.concatenate(
        [srcs, (jnp.arange(npad, dtype=jnp.int32) * 97) % NV]).reshape(
            16 * EBT, EB)
    dsts2 = jnp.concatenate(
        [dsts, DUMMY + (jnp.arange(npad, dtype=jnp.int32) % 8)]).reshape(
            16 * EBT, EB)

    idx4, w4 = _prep_call(verts_t)
    tap_idx = jnp.pad(idx4.transpose(1, 0, 2).reshape(4, NV),
                      ((0, 0), (0, NVP - NV)))
    tap_w = jnp.pad(w4.transpose(1, 0, 2).reshape(4, NV),
                    ((0, 0), (0, NVP - NV)))
    tap_idx = tap_idx.reshape(4, NVP // SB, SB).transpose(1, 0, 2).reshape(
        NVP // SB, 4 * SB)
    tap_w = tap_w.reshape(4, NVP // SB, SB).transpose(1, 0, 2).reshape(
        NVP // SB, 4 * SB)
    tap_w = jnp.pad(tap_w, ((0, 0), (0, SB)))

    pp = _pix_call(img_t, bneck_w.T)
    table = jnp.concatenate([pp, sem_t], axis=1)

    st = _sample_call(tap_idx, tap_w, table, bneck_b)
    va_sem = st[:, :HID]
    tex = st[:, HID:]

    x = jnp.concatenate([va_sem, pos_pad, tex, prev_pad], axis=1)
    h = None
    for (w0, b0, w1, b1) in ((g0_w0, g0_b0, g0_w1, g0_b1),
                             (g1_w0, g1_b0, g1_w1, g1_b1),
                             (g2_w0, g2_b0, g2_w1, g2_b1)):
        vw0, vw1 = _mm2_call(x, w0.T, w1.T, b0[None], b1[None])
        h = _agg_call(srcs2, dsts2, vw1.reshape(NVP * NCH, CW),
                      vw0.reshape(NVP * NCH, CW)).reshape(NVP, HID)
        x = jnp.concatenate([h, pos_pad, tex], axis=1)

    finp = _fin_call(x, vsem_w.T, vsem_b[None], tex)
    final_textures = finp[:NV].reshape(-1, V, NC)
    return final_textures, h[:NV]


# speedup vs baseline: 2.0860x; 1.2594x over previous
import jax
import jax.numpy as jnp
from jax import lax
from jax.experimental import pallas as pl
from jax.experimental.pallas import tpu as pltpu
from jax.experimental.pallas import tpu_sc as plsc

N, V, IMG_DIM, HID, VFD, NC, H, W = 4, 10000, 512, 512, 128, 32, 64, 64
NV = N * V
PIX = H * W
TD = HID + NC
FC = HID // NC
NVP = 40960
VPW = NVP // 32
SB = 32
SBT = VPW // SB
E = 120000
NE2 = 2 * E
EB = 128
EBT = 120
NB = 12
NE2P = 16 * EB * EBT
DUMMY = NVP - 8
CW = 16
NCH = HID // CW
ZR = NVP // 16
EPB = 256

def _mesh():
    return plsc.VectorSubcoreMesh(core_axis_name="c", subcore_axis_name="s",
                                  num_cores=2, num_subcores=16)


def _prep_body(vt_ref, idx_ref, w_ref):
    n = pl.program_id(0)
    gx = (vt_ref[0, 0] + 1.0) * 0.5 * (W - 1)
    gy = (vt_ref[0, 1] + 1.0) * 0.5 * (H - 1)
    x0 = jnp.floor(gx)
    x1 = x0 + 1.0
    y0 = jnp.floor(gy)
    y1 = y0 + 1.0
    wx1 = gx - x0
    wx0 = 1.0 - wx1
    wy1 = gy - y0
    wy0 = 1.0 - wy1
    base = n * PIX
    taps = [(x0, y0, wx0 * wy0), (x1, y0, wx1 * wy0),
            (x0, y1, wx0 * wy1), (x1, y1, wx1 * wy1)]
    for t, (xi, yi, wt) in enumerate(taps):
        valid = ((xi >= 0) & (xi <= W - 1) & (yi >= 0) & (yi <= H - 1))
        xci = jnp.clip(xi, 0, W - 1).astype(jnp.int32)
        yci = jnp.clip(yi, 0, H - 1).astype(jnp.int32)
        idx_ref[0, t] = base + yci * W + xci
        w_ref[0, t] = wt * valid.astype(jnp.float32)


def _prep_call(verts_t):
    return pl.pallas_call(
        _prep_body,
        grid=(N,),
        in_specs=[
            pl.BlockSpec((1, 3, V), lambda n: (n, 0, 0)),
        ],
        out_specs=[
            pl.BlockSpec((1, 4, V), lambda n: (n, 0, 0)),
            pl.BlockSpec((1, 4, V), lambda n: (n, 0, 0)),
        ],
        out_shape=[
            jax.ShapeDtypeStruct((N, 4, V), jnp.int32),
            jax.ShapeDtypeStruct((N, 4, V), jnp.float32),
        ],
    )(verts_t)


def _pix_body(a_ref, w_ref, o_ref):
    o_ref[...] = jnp.dot(a_ref[...], w_ref[...],
                         preferred_element_type=jnp.float32)


def _pix_call(img_t, wt):
    m = N * PIX
    return pl.pallas_call(
        _pix_body,
        grid=(m // 512,),
        in_specs=[
            pl.BlockSpec((512, IMG_DIM), lambda i: (i, 0)),
            pl.BlockSpec((IMG_DIM, HID), lambda i: (0, 0)),
        ],
        out_specs=pl.BlockSpec((512, HID), lambda i: (i, 0)),
        out_shape=jax.ShapeDtypeStruct((m, HID), jnp.float32),
    )(img_t, wt)


def _sample_body(tap_idx, tap_w, table, bias, out, idxb, wb, rows, outb,
                 biasb, sem):
    cid = lax.axis_index("c")
    sid = lax.axis_index("s")
    wid = sid * 2 + cid
    bbase = wid * SBT
    pltpu.sync_copy(bias, biasb)

    @pl.loop(0, SBT)
    def _batch(b):
        bg = bbase + b
        pltpu.sync_copy(tap_idx.at[bg], idxb)
        pltpu.sync_copy(tap_w.at[bg], wb)
        for t in range(4):
            pltpu.async_copy(table.at[idxb.at[pl.ds(t * SB, SB)]],
                             rows.at[t], sem).wait()

        @pl.loop(0, SB)
        def _v(v):
            w0 = wb[pl.ds(v, 16)][0]
            w1 = wb[pl.ds(v + SB, 16)][0]
            w2 = wb[pl.ds(v + 2 * SB, 16)][0]
            w3 = wb[pl.ds(v + 3 * SB, 16)][0]
            for c in range(TD // 16):
                sl = pl.ds(c * 16, 16)
                acc = (rows[0, v, sl] * w0 + rows[1, v, sl] * w1
                       + rows[2, v, sl] * w2 + rows[3, v, sl] * w3)
                if c < HID // 16:
                    acc = jnp.maximum(acc + biasb[sl], 0.0)
                outb[v, sl] = acc

        pltpu.sync_copy(outb, out.at[pl.ds(bg * SB, SB)])


def _sample_call(tap_idx, tap_w, table, bias):
    return pl.kernel(
        _sample_body,
        out_type=jax.ShapeDtypeStruct((NVP, TD), jnp.float32),
        mesh=_mesh(),
        compiler_params=pltpu.CompilerParams(use_tc_tiling_on_sc=False),
        scratch_types=[
            pltpu.VMEM((4 * SB,), jnp.int32),
            pltpu.VMEM((4 * SB + SB,), jnp.float32),
            pltpu.VMEM((4, SB, TD), jnp.float32),
            pltpu.VMEM((SB, TD), jnp.float32),
            pltpu.VMEM((HID,), jnp.float32),
            pltpu.SemaphoreType.DMA,
        ],
    )(tap_idx, tap_w, table, bias)


def _agg_body(srcs2, dsts2, vw1r, vw0r, h_out, srcb, dstb, srcadj,
              rows, abuf, bbuf, zbuf, acc, sem):
    cid = lax.axis_index("c")
    sid = lax.axis_index("s")
    pltpu.sync_copy(srcs2.at[pl.ds(sid * EBT, EBT)], srcb)
    pltpu.sync_copy(dsts2.at[pl.ds(sid * EBT, EBT)], dstb)
    zv = jnp.zeros((16,), jnp.float32)

    @pl.loop(0, EPB)
    def _z(i):
        zbuf[i] = zv

    @pl.loop(0, NCH // 2)
    def _chunk(cc):
        chunk = cid * (NCH // 2) + cc
        chunkv = jnp.zeros((16,), jnp.int32) + chunk

        @pl.loop(0, ZR // EPB)
        def _zero(p):
            pltpu.sync_copy(zbuf, acc.at[pl.ds(sid * ZR + p * EPB, EPB)])

        plsc.subcore_barrier()

        @pl.loop(0, EBT // NB)
        def _grp(g):
            j0 = g * NB
            for b in range(NB):
                for k in range(EB // 16):
                    sl = pl.ds(k * 16, 16)
                    srcadj[b, sl] = srcb[j0 + b, sl] * NCH + chunkv
            gs = [pltpu.async_copy(vw1r.at[srcadj.at[b]], rows.at[b], sem)
                  for b in range(NB)]
            for h in gs:
                h.wait()
            ss = [pltpu.async_copy(rows.at[b], acc.at[dstb.at[j0 + b]], sem,
                                   add=True)
                  for b in range(NB)]
            for h in ss:
                h.wait()

        plsc.subcore_barrier()

        @pl.loop(0, ZR // EPB)
        def _ep(p):
            r0 = sid * ZR + p * EPB
            cbase = chunk * CW
            pltpu.sync_copy(acc.at[pl.ds(r0, EPB)], abuf)
            pltpu.async_copy(vw0r.at[pl.ds(r0, EPB), pl.ds(cbase, CW)],
                             bbuf, sem).wait()

            @pl.loop(0, EPB)
            def _h(i):
                abuf[i] = jnp.maximum(abuf[i] + bbuf[i], 0.0)

            pltpu.sync_copy(abuf, h_out.at[pl.ds(r0, EPB), pl.ds(cbase, CW)])

        plsc.subcore_barrier()


def _agg_call(srcs2, dsts2, vw1, vw0):
    return pl.kernel(
        _agg_body,
        out_type=jax.ShapeDtypeStruct((NVP, HID), jnp.float32),
        mesh=_mesh(),
        compiler_params=pltpu.CompilerParams(use_tc_tiling_on_sc=False),
        scratch_types=[
            pltpu.VMEM((EBT, EB), jnp.int32),
            pltpu.VMEM((EBT, EB), jnp.int32),
            pltpu.VMEM((NB, EB), jnp.int32),
            pltpu.VMEM((NB, EB, CW), jnp.float32),
            pltpu.VMEM((EPB, CW), jnp.float32),
            pltpu.VMEM((EPB, CW), jnp.float32),
            pltpu.VMEM((EPB, CW), jnp.float32),
            pltpu.VMEM_SHARED((NVP, CW), jnp.float32),
            pltpu.SemaphoreType.DMA,
        ],
    )(srcs2, dsts2, vw1.reshape(NVP * NCH, CW), vw0)


def _mm2_body(a_ref, w0_ref, w1_ref, b0_ref, b1_ref, o0_ref, o1_ref):
    a = a_ref[...]
    o0_ref[...] = (jnp.dot(a, w0_ref[...], preferred_element_type=jnp.float32)
                   + b0_ref[...])
    o1_ref[...] = (jnp.dot(a, w1_ref[...], preferred_element_type=jnp.float32)
                   + b1_ref[...])


def _mm2_call(x, w0t, w1t, b0, b1):
    k = x.shape[1]
    return pl.pallas_call(
        _mm2_body,
        grid=(NVP // 512,),
        in_specs=[
            pl.BlockSpec((512, k), lambda i: (i, 0)),
            pl.BlockSpec((k, HID), lambda i: (0, 0)),
            pl.BlockSpec((k, HID), lambda i: (0, 0)),
            pl.BlockSpec((1, HID), lambda i: (0, 0)),
            pl.BlockSpec((1, HID), lambda i: (0, 0)),
        ],
        out_specs=[
            pl.BlockSpec((512, HID), lambda i: (i, 0)),
            pl.BlockSpec((512, HID), lambda i: (i, 0)),
        ],
        out_shape=[
            jax.ShapeDtypeStruct((NVP, HID), jnp.float32),
            jax.ShapeDtypeStruct((NVP, HID), jnp.float32),
        ],
    )(x, w0t, w1t, b0, b1)


def _fin_body(a_ref, w_ref, b_ref, t_ref, o_ref):
    o_ref[...] = (jnp.dot(a_ref[...], w_ref[...],
                          preferred_element_type=jnp.float32)
                  + b_ref[...] + t_ref[...])


def _fin_call(x, wt, b, tex):
    k = x.shape[1]
    return pl.pallas_call(
        _fin_body,
        grid=(NVP // 512,),
        in_specs=[
            pl.BlockSpec((512, k), lambda i: (i, 0)),
            pl.BlockSpec((k, NC), lambda i: (0, 0)),
            pl.BlockSpec((1, NC), lambda i: (0, 0)),
            pl.BlockSpec((512, NC), lambda i: (i, 0)),
        ],
        out_specs=pl.BlockSpec((512, NC), lambda i: (i, 0)),
        out_shape=jax.ShapeDtypeStruct((NVP, NC), jnp.float32),
    )(x, wt, b, tex)


def kernel(img_feats, verts_padded, edges_packed, vert_feats_prev, sem_2d, P,
           bneck_w, bneck_b, vsem_w, vsem_b,
           g0_w0, g0_b0, g0_w1, g0_b1,
           g1_w0, g1_b0, g1_w1, g1_b1,
           g2_w0, g2_b0, g2_w1, g2_b1):
    ones = jnp.ones((N, V, 1), verts_padded.dtype)
    verts_hom = jnp.concatenate([verts_padded, ones], axis=2)
    verts_cam = jnp.einsum('nvi,nji->nvj', verts_hom, P)
    wcl = verts_cam[:, :, 3:]
    eps = 1e-1
    wcl = jnp.where(jnp.abs(wcl) < eps, jnp.where(wcl >= 0, eps, -eps), wcl)
    verts_t = (verts_cam[:, :, :3] / wcl).transpose(0, 2, 1)
    img_t = img_feats.transpose(0, 2, 3, 1).reshape(N * PIX, IMG_DIM)
    sem_t = sem_2d.transpose(0, 2, 3, 1).reshape(N * PIX, NC)
    pos = verts_padded.reshape(NV, 3)
    pos_pad = jnp.pad(pos, ((0, NVP - NV), (0, 0)))
    prev_pad = jnp.pad(vert_feats_prev, ((0, NVP - NV), (0, 0)))

    srcs = jnp.concatenate([edges_packed[:, 1], edges_packed[:, 0]])
    dsts = jnp.concatenate([edges_packed[:, 0], edges_packed[:, 1]])
    npad = NE2P - NE2
    srcs2 = jnp.concatenate(
        [srcs, (jnp.arange(npad, dtype=jnp.int32) * 97) % NV]).reshape(
            16 * EBT, EB)
    dsts2 = jnp.concatenate(
        [dsts, DUMMY + (jnp.arange(npad, dtype=jnp.int32) % 8)]).reshape(
            16 * EBT, EB)

    idx4, w4 = _prep_call(verts_t)
    tap_idx = jnp.pad(idx4.transpose(1, 0, 2).reshape(4, NV),
                      ((0, 0), (0, NVP - NV)))
    tap_w = jnp.pad(w4.transpose(1, 0, 2).reshape(4, NV),
                    ((0, 0), (0, NVP - NV)))
    tap_idx = tap_idx.reshape(4, NVP // SB, SB).transpose(1, 0, 2).reshape(
        NVP // SB, 4 * SB)
    tap_w = tap_w.reshape(4, NVP // SB, SB).transpose(1, 0, 2).reshape(
        NVP // SB, 4 * SB)
    tap_w = jnp.pad(tap_w, ((0, 0), (0, SB)))

    pp = _pix_call(img_t, bneck_w.T)
    table = jnp.concatenate([pp, sem_t], axis=1)

    st = _sample_call(tap_idx, tap_w, table, bneck_b)
    va_sem = st[:, :HID]
    tex = st[:, HID:]

    x = jnp.concatenate([va_sem, pos_pad, tex, prev_pad], axis=1)
    h = None
    for (w0, b0, w1, b1) in ((g0_w0, g0_b0, g0_w1, g0_b1),
                             (g1_w0, g1_b0, g1_w1, g1_b1),
                             (g2_w0, g2_b0, g2_w1, g2_b1)):
        vw0, vw1 = _mm2_call(x, w0.T, w1.T, b0[None], b1[None])
        h = _agg_call(srcs2, dsts2, vw1, vw0)
        x = jnp.concatenate([h, pos_pad, tex], axis=1)

    finp = _fin_call(x, vsem_w.T, vsem_b[None], tex)
    final_textures = finp[:NV].reshape(-1, V, NC)
    return final_textures, h[:NV]
